# baseline (speedup 1.0000x reference)
import jax
import jax.numpy as jnp
from jax.experimental import pallas as pl
from jax.experimental.pallas import tpu as pltpu

_K = 128
_ROWS = 128
_N = 8192
_NBLK = _N // _K
_DISTS = (64, 32, 16, 8, 4, 2, 1)


def _cmp_ex(w, d, keep_max):
    up = (jax.lax.broadcasted_iota(jnp.int32, (1, _K), 1) & d) == 0
    partner = jnp.where(up, pltpu.roll(w, _K - d, 1), pltpu.roll(w, d, 1))
    return jnp.where(keep_max, jnp.maximum(w, partner),
                     jnp.minimum(w, partner))


def _topk_body(x_ref, o_ref):
    half = _NBLK // 2
    rows = x_ref.shape[0]
    w = jnp.stack(
        [x_ref[:, b * _K:(b + 1) * _K] for b in range(half)]
        + [-x_ref[:, b * _K:(b + 1) * _K] for b in range(half, _NBLK)],
        axis=1,
    ).reshape(rows * _NBLK, _K)

    lane = jax.lax.broadcasted_iota(jnp.int32, (1, _K), 1)
    up_of = {d: (lane & d) == 0 for d in _DISTS}

    m = 2
    while m <= _K:
        stage_desc = (lane & m) == 0
        d = m // 2
        while d >= 1:
            w = _cmp_ex(w, d, up_of[d] == stage_desc)
            d //= 2
        m *= 2

    nblk = _NBLK
    while nblk > 1:
        nblk //= 2
        v = w.reshape(rows, 2 * nblk, _K)
        w = jnp.maximum(v[:, :nblk], -v[:, nblk:])
        if nblk > 1:
            w = jnp.concatenate([w[:, :nblk // 2], -w[:, nblk // 2:]],
                                axis=1)
        w = w.reshape(rows * nblk, _K)
        for d in _DISTS:
            w = _cmp_ex(w, d, up_of[d])

    o_ref[...] = w


def kernel(x):
    return pl.pallas_call(
        _topk_body,
        out_shape=jax.ShapeDtypeStruct((_ROWS, _K), jnp.float32),
        in_specs=[pl.BlockSpec((_ROWS, _N), lambda: (0, 0))],
        out_specs=pl.BlockSpec((_ROWS, _K), lambda: (0, 0)),
    )(x)


# speedup vs baseline: 1.7820x; 1.7820x over previous
import jax
import jax.numpy as jnp
from jax.experimental import pallas as pl

_K = 128
_ROWS = 128
_N = 8192
_NBLK = _N // _K
_DISTS = (64, 32, 16, 8, 4, 2, 1)


def _cmp_ex(w, d, keep_max):
    perm = jax.lax.broadcasted_iota(jnp.int32, w.shape, 1) ^ d
    partner = jnp.take_along_axis(w, perm, axis=1)
    return jnp.where(keep_max, jnp.maximum(w, partner),
                     jnp.minimum(w, partner))


def _topk_body(x_ref, o_ref):
    half = _NBLK // 2
    rows = x_ref.shape[0]
    w = jnp.stack(
        [x_ref[:, b * _K:(b + 1) * _K] for b in range(half)]
        + [-x_ref[:, b * _K:(b + 1) * _K] for b in range(half, _NBLK)],
        axis=1,
    ).reshape(rows * _NBLK, _K)

    lane = jax.lax.broadcasted_iota(jnp.int32, (1, _K), 1)
    up_of = {d: (lane & d) == 0 for d in _DISTS}

    m = 2
    while m <= _K:
        stage_desc = (lane & m) == 0
        d = m // 2
        while d >= 1:
            w = _cmp_ex(w, d, up_of[d] == stage_desc)
            d //= 2
        m *= 2

    nblk = _NBLK
    while nblk > 1:
        nblk //= 2
        v = w.reshape(rows, 2 * nblk, _K)
        w = jnp.maximum(v[:, :nblk], -v[:, nblk:])
        if nblk > 1:
            w = jnp.concatenate([w[:, :nblk // 2], -w[:, nblk // 2:]],
                                axis=1)
        w = w.reshape(rows * nblk, _K)
        for d in _DISTS:
            w = _cmp_ex(w, d, up_of[d])

    o_ref[...] = w


def kernel(x):
    return pl.pallas_call(
        _topk_body,
        out_shape=jax.ShapeDtypeStruct((_ROWS, _K), jnp.float32),
        in_specs=[pl.BlockSpec((_ROWS, _N), lambda: (0, 0))],
        out_specs=pl.BlockSpec((_ROWS, _K), lambda: (0, 0)),
    )(x)


# speedup vs baseline: 1.7849x; 1.0016x over previous
import jax
import jax.numpy as jnp
from jax.experimental import pallas as pl

_K = 128
_ROWS = 128
_N = 8192
_NBLK = _N // _K
_DISTS = (64, 32, 16, 8, 4, 2, 1)


def _cmp_ex(w, d, keep_max):
    perm = jax.lax.broadcasted_iota(jnp.int32, w.shape, 1) ^ d
    partner = jnp.take_along_axis(w, perm, axis=1)
    return jnp.where(keep_max, jnp.maximum(w, partner),
                     jnp.minimum(w, partner))


def _topk_body(x_ref, o_ref):
    half = _NBLK // 2
    rows = x_ref.shape[0]
    w = jnp.stack(
        [x_ref[:, b * _K:(b + 1) * _K] for b in range(half)]
        + [-x_ref[:, b * _K:(b + 1) * _K] for b in range(half, _NBLK)],
        axis=1,
    ).reshape(rows * _NBLK, _K)

    lane = jax.lax.broadcasted_iota(jnp.int32, (1, _K), 1)
    up_of = {d: (lane & d) == 0 for d in _DISTS}

    m = 2
    while m <= _K:
        stage_desc = (lane & m) == 0
        d = m // 2
        while d >= 1:
            w = _cmp_ex(w, d, up_of[d] == stage_desc)
            d //= 2
        m *= 2

    nblk = _NBLK
    while nblk > 1:
        nblk //= 2
        v = w.reshape(rows, 2 * nblk, _K)
        w = jnp.maximum(v[:, :nblk], -v[:, nblk:])
        if nblk > 1:
            w = jnp.concatenate([w[:, :nblk // 2], -w[:, nblk // 2:]],
                                axis=1)
        w = w.reshape(rows * nblk, _K)
        for d in _DISTS:
            w = _cmp_ex(w, d, up_of[d])

    o_ref[...] = w


def kernel(x):
    return pl.pallas_call(
        _topk_body,
        grid=(2,),
        out_shape=jax.ShapeDtypeStruct((_ROWS, _K), jnp.float32),
        in_specs=[pl.BlockSpec((_ROWS // 2, _N), lambda i: (i, 0))],
        out_specs=pl.BlockSpec((_ROWS // 2, _K), lambda i: (i, 0)),
    )(x)
